# baseline (speedup 1.0000x reference)
import functools

import jax
import jax.numpy as jnp
from jax import lax
from jax.experimental import pallas as pl
from jax.experimental.pallas import tpu as pltpu
from jax.experimental.pallas import tpu_sc as plsc

F32 = jnp.float32
NCORES = 2
NSUB = 16
CK = 512
CKR = CK // 128
TBL = 16
QW = 32
NQ = 4
G_SEG = 64


def _round_up(v, m):
    return (v + m - 1) // m * m


def _edge_sweep(src_hbm, dst_hbm, tbl, acc, sidx, didx, r0, r1, sem0, sem1,
                row_base, groups):
    def fire(j, rbuf, sem):
        pltpu.async_copy(tbl.at[sidx.at[j]], rbuf, sem)

    def wait(j, rbuf, sem):
        pltpu.make_async_copy(tbl.at[sidx.at[j]], rbuf, sem).wait()

    def scat(j, rbuf):
        pltpu.sync_copy(rbuf, acc.at[didx.at[j]], add=True)

    @pl.loop(0, groups)
    def _(g):
        row0 = row_base + g * 8
        pltpu.sync_copy(src_hbm.at[pl.ds(row0, 8)], sidx)
        pltpu.sync_copy(dst_hbm.at[pl.ds(row0, 8)], didx)
        fire(0, r0, sem0)

        @pl.loop(0, 4)
        def _(p):
            j0 = 2 * p
            fire(j0 + 1, r1, sem1)
            wait(j0, r0, sem0)
            scat(j0, r0)

            @pl.when(p < 3)
            def _():
                fire(j0 + 2, r0, sem0)

            wait(j0 + 1, r1, sem1)
            scat(j0 + 1, r1)


def _deg_kernel(npad, erows, interpret=False):
    rz = npad // NSUB
    chunks = erows // (NCORES * NSUB * CKR)
    mesh = plsc.VectorSubcoreMesh(core_axis_name="c", subcore_axis_name="s",
                                  num_cores=NCORES, num_subcores=NSUB)

    @functools.partial(
        pl.kernel,
        out_type=jax.ShapeDtypeStruct((NCORES * npad, TBL), F32),
        mesh=mesh,
        scratch_types=[
            pltpu.VMEM((CKR, 128), jnp.int32),
            pltpu.VMEM((128, TBL), F32),
            pltpu.VMEM_SHARED((npad, TBL), F32),
        ],
        compiler_params=pltpu.CompilerParams(use_tc_tiling_on_sc=False),
        interpret=interpret,
    )
    def k(dst_hbm, zeros_hbm, out_hbm, didx, ones_v, acc):
        c = lax.axis_index("c")
        s = lax.axis_index("s")
        wid = c * NSUB + s

        @pl.loop(0, 128)
        def _(r):
            ones_v[r, :] = jnp.full((TBL,), 1.0, F32)

        pltpu.sync_copy(zeros_hbm.at[pl.ds(s * rz, rz)],
                        acc.at[pl.ds(s * rz, rz)])
        plsc.subcore_barrier()

        @pl.loop(0, chunks)
        def _(kk):
            row0 = (wid * chunks + kk) * CKR
            pltpu.sync_copy(dst_hbm.at[pl.ds(row0, CKR)], didx)
            for j in range(CKR):
                pltpu.sync_copy(ones_v.at[pl.ds(0, 128)],
                                acc.at[didx.at[j]], add=True)

        plsc.subcore_barrier()
        pltpu.sync_copy(acc.at[pl.ds(s * rz, rz)],
                        out_hbm.at[pl.ds(c * npad + s * rz, rz)])

    return k


def _agg1_kernel(npad, erows, interpret=False):
    rz = npad // NSUB
    groups = erows // (NCORES * NSUB * 8)
    mesh = plsc.VectorSubcoreMesh(core_axis_name="c", subcore_axis_name="s",
                                  num_cores=NCORES, num_subcores=NSUB)

    @functools.partial(
        pl.kernel,
        out_type=jax.ShapeDtypeStruct((NCORES * npad, TBL), F32),
        mesh=mesh,
        scratch_types=[
            pltpu.VMEM((8, 128), jnp.int32),
            pltpu.VMEM((8, 128), jnp.int32),
            pltpu.VMEM((128, TBL), F32),
            pltpu.VMEM((128, TBL), F32),
            pltpu.VMEM_SHARED((npad, TBL), F32),
            pltpu.SemaphoreType.DMA,
            pltpu.SemaphoreType.DMA,
        ],
        compiler_params=pltpu.CompilerParams(use_tc_tiling_on_sc=False),
        interpret=interpret,
    )
    def k(src_hbm, dst_hbm, tbl_hbm, zeros_hbm, out_hbm, sidx, didx, r0, r1,
          acc, sem0, sem1):
        c = lax.axis_index("c")
        s = lax.axis_index("s")
        wid = c * NSUB + s

        pltpu.sync_copy(zeros_hbm.at[pl.ds(s * rz, rz)],
                        acc.at[pl.ds(s * rz, rz)])
        plsc.subcore_barrier()
        _edge_sweep(src_hbm, dst_hbm, tbl_hbm, acc, sidx, didx, r0, r1,
                    sem0, sem1, wid * groups * 8, groups)
        plsc.subcore_barrier()
        pltpu.sync_copy(acc.at[pl.ds(s * rz, rz)],
                        out_hbm.at[pl.ds(c * npad + s * rz, rz)])

    return k


def _agg2_kernel(npad, erows, interpret=False):
    rz = npad // NSUB
    groups = erows // (NSUB * 8)
    mesh = plsc.VectorSubcoreMesh(core_axis_name="c", subcore_axis_name="s",
                                  num_cores=NCORES, num_subcores=NSUB)

    @functools.partial(
        pl.kernel,
        out_type=jax.ShapeDtypeStruct((NQ * npad, QW), F32),
        mesh=mesh,
        scratch_types=[
            pltpu.VMEM((8, 128), jnp.int32),
            pltpu.VMEM((8, 128), jnp.int32),
            pltpu.VMEM((128, QW), F32),
            pltpu.VMEM((128, QW), F32),
            pltpu.VMEM_SHARED((npad, QW), F32),
            pltpu.SemaphoreType.DMA,
            pltpu.SemaphoreType.DMA,
        ],
        compiler_params=pltpu.CompilerParams(use_tc_tiling_on_sc=False),
        interpret=interpret,
    )
    def k(src_hbm, dst_hbm, g0, g1, g2, g3, zeros_hbm, out_hbm, sidx, didx,
          r0, r1, acc, sem0, sem1):
        c = lax.axis_index("c")
        s = lax.axis_index("s")
        gq_refs = [g0, g1, g2, g3]

        for q in range(NQ):
            gq = gq_refs[q]

            @pl.when(c == q // 2)
            def _():
                pltpu.sync_copy(zeros_hbm.at[pl.ds(s * rz, rz)],
                                acc.at[pl.ds(s * rz, rz)])
                plsc.subcore_barrier()
                _edge_sweep(src_hbm, dst_hbm, gq, acc, sidx, didx, r0,
                            r1, sem0, sem1, s * groups * 8, groups)
                plsc.subcore_barrier()
                pltpu.sync_copy(acc.at[pl.ds(s * rz, rz)],
                                out_hbm.at[pl.ds(q * npad + s * rz, rz)])
                plsc.subcore_barrier()

    return k


def _dinv_call(deg16, xp, npad, nfeat, interpret=False):
    bn = npad // NSUB

    def body(deg_ref, x_ref, dinv_ref, tbl_ref):
        deg = deg_ref[0, :, 0:1] + deg_ref[1, :, 0:1] + 1.0
        dinv = lax.rsqrt(deg)
        dinv_ref[...] = dinv
        gx = dinv * x_ref[...]
        tbl_ref[...] = jnp.concatenate(
            [gx, jnp.zeros((bn, TBL - nfeat), F32)], axis=1)

    return pl.pallas_call(
        body,
        grid=(NSUB,),
        in_specs=[
            pl.BlockSpec((NCORES, bn, TBL), lambda i: (0, i, 0)),
            pl.BlockSpec((bn, nfeat), lambda i: (i, 0)),
        ],
        out_specs=[
            pl.BlockSpec((bn, 1), lambda i: (i, 0)),
            pl.BlockSpec((bn, TBL), lambda i: (i, 0)),
        ],
        out_shape=[
            jax.ShapeDtypeStruct((npad, 1), F32),
            jax.ShapeDtypeStruct((npad, TBL), F32),
        ],
        interpret=interpret,
    )(deg16, xp)


def _dense_call(aggx, tbl, dinv, W1, b1, W2, npad, nfeat, interpret=False):
    bn = npad // NSUB

    def body(aggx_ref, tbl_ref, dinv_ref, w1_ref, b1_ref, w2_ref,
             g0_ref, g1_ref, g2_ref, g3_ref):
        dinv = dinv_ref[...]
        aggp = aggx_ref[0, :, 0:nfeat] + aggx_ref[1, :, 0:nfeat]
        ax = dinv * (aggp + tbl_ref[:, 0:nfeat])
        h1 = jnp.maximum(
            jnp.dot(ax, w1_ref[...], preferred_element_type=F32)
            + b1_ref[...], 0.0)
        m = jnp.dot(h1, w2_ref[...], preferred_element_type=F32)
        g = dinv * m
        g0_ref[...] = g[:, 0:QW]
        g1_ref[...] = g[:, QW:2 * QW]
        g2_ref[...] = g[:, 2 * QW:3 * QW]
        g3_ref[...] = g[:, 3 * QW:4 * QW]

    gspec = pl.BlockSpec((bn, QW), lambda i: (i, 0))
    gshape = jax.ShapeDtypeStruct((npad, QW), F32)
    return pl.pallas_call(
        body,
        grid=(NSUB,),
        in_specs=[
            pl.BlockSpec((NCORES, bn, TBL), lambda i: (0, i, 0)),
            pl.BlockSpec((bn, TBL), lambda i: (i, 0)),
            pl.BlockSpec((bn, 1), lambda i: (i, 0)),
            pl.BlockSpec((nfeat, 128), lambda i: (0, 0)),
            pl.BlockSpec((1, 128), lambda i: (0, 0)),
            pl.BlockSpec((128, 128), lambda i: (0, 0)),
        ],
        out_specs=[gspec, gspec, gspec, gspec],
        out_shape=[gshape, gshape, gshape, gshape],
        interpret=interpret,
    )(aggx, tbl, dinv, W1, b1, W2)


def _final_call(agg, g0, g1, g2, g3, dinv, b2, batch_row, Wfc, bfc, npad,
                interpret=False):
    bn = npad // NSUB
    nb = NSUB

    def body(a0_ref, a1_ref, a2_ref, a3_ref, g0_ref, g1_ref, g2_ref, g3_ref,
             dinv_ref, b2_ref, batch_ref, wfc_ref, bfc_ref, out_ref,
             pooled_acc, cnt_acc):
        i = pl.program_id(0)

        @pl.when(i == 0)
        def _():
            pooled_acc[...] = jnp.zeros((G_SEG, 128), F32)
            cnt_acc[...] = jnp.zeros((G_SEG, 128), F32)
            out_ref[...] = jnp.zeros((G_SEG, 10), F32)

        dinv = dinv_ref[...]
        h2 = jnp.concatenate(
            [jnp.maximum(dinv * (a_ref[...] + g_ref[...])
                         + b2_ref[:, q * QW:(q + 1) * QW], 0.0)
             for q, (a_ref, g_ref) in enumerate(
                 zip([a0_ref, a1_ref, a2_ref, a3_ref],
                     [g0_ref, g1_ref, g2_ref, g3_ref]))], axis=1)

        seg = batch_ref[...]
        onehot_t = (lax.broadcasted_iota(jnp.int32, (G_SEG, bn), 0)
                    == seg).astype(F32)
        pooled_acc[...] += lax.dot_general(
            onehot_t, h2, (((1,), (0,)), ((), ())),
            preferred_element_type=F32)
        cnt_acc[...] += lax.dot_general(
            onehot_t, jnp.ones((bn, 128), F32), (((1,), (0,)), ((), ())),
            preferred_element_type=F32)

        @pl.when(i == nb - 1)
        def _():
            mean = pooled_acc[...] / jnp.maximum(cnt_acc[...], 1.0)
            out_ref[...] = jnp.dot(mean, wfc_ref[...],
                                   preferred_element_type=F32) + bfc_ref[...]

    aspec = lambda q: pl.BlockSpec((bn, QW),
                                   lambda i, q=q: (q * NSUB + i, 0))
    gspec = pl.BlockSpec((bn, QW), lambda i: (i, 0))
    return pl.pallas_call(
        body,
        grid=(nb,),
        in_specs=[
            aspec(0), aspec(1), aspec(2), aspec(3),
            gspec, gspec, gspec, gspec,
            pl.BlockSpec((bn, 1), lambda i: (i, 0)),
            pl.BlockSpec((1, 128), lambda i: (0, 0)),
            pl.BlockSpec((1, bn), lambda i: (0, i)),
            pl.BlockSpec((128, 10), lambda i: (0, 0)),
            pl.BlockSpec((1, 10), lambda i: (0, 0)),
        ],
        out_specs=pl.BlockSpec((G_SEG, 10), lambda i: (0, 0)),
        out_shape=jax.ShapeDtypeStruct((G_SEG, 10), F32),
        scratch_shapes=[
            pltpu.VMEM((G_SEG, 128), F32),
            pltpu.VMEM((G_SEG, 128), F32),
        ],
        interpret=interpret,
    )(agg, agg, agg, agg, g0, g1, g2, g3, dinv, b2, batch_row, Wfc, bfc)


def _run(x, edge_index, batch, W1, b1, W2, b2, Wfc, bfc,
         interpret_sc=False, interpret_tc=False):
    n, nfeat = x.shape
    e = edge_index.shape[1]
    npad = _round_up(n + 1, NSUB * 128)
    epad = _round_up(e, NCORES * NSUB * 8 * 128)
    erows = epad // 128

    src = edge_index[0]
    dst = edge_index[1]
    padlen = epad - e
    fill = jnp.full((padlen,), n, jnp.int32)
    srcr = jnp.concatenate([src, fill]).reshape(erows, 128)
    dstr = jnp.concatenate([dst, fill]).reshape(erows, 128)
    xp = jnp.pad(x, ((0, npad - n), (0, 0)))
    batch_row = jnp.pad(batch, (0, npad - n),
                        constant_values=G_SEG).reshape(1, npad)
    zeros16 = jnp.zeros((npad, TBL), F32)
    zeros32 = jnp.zeros((npad, QW), F32)

    degp = _deg_kernel(npad, erows, interpret_sc)(dstr, zeros16)
    degp = degp.reshape(NCORES, npad, TBL)
    dinv, tbl = _dinv_call(degp, xp, npad, nfeat, interpret_tc)
    aggx = _agg1_kernel(npad, erows, interpret_sc)(srcr, dstr, tbl, zeros16)
    aggx = aggx.reshape(NCORES, npad, TBL)
    g0, g1, g2, g3 = _dense_call(aggx, tbl, dinv, W1, b1.reshape(1, 128), W2,
                                 npad, nfeat, interpret_tc)
    agg = _agg2_kernel(npad, erows, interpret_sc)(srcr, dstr, g0, g1, g2, g3,
                                                  zeros32)
    out = _final_call(agg, g0, g1, g2, g3, dinv, b2.reshape(1, 128),
                      batch_row, Wfc, bfc.reshape(1, 10), npad, interpret_tc)
    return out


def kernel(x, edge_index, batch, W1, b1, W2, b2, Wfc, bfc):
    return _run(x, edge_index, batch, W1, b1, W2, b2, Wfc, bfc)


# speedup vs baseline: 1.0297x; 1.0297x over previous
import functools

import jax
import jax.numpy as jnp
from jax import lax
from jax.experimental import pallas as pl
from jax.experimental.pallas import tpu as pltpu
from jax.experimental.pallas import tpu_sc as plsc

F32 = jnp.float32
NCORES = 2
NSUB = 16
CK = 512
CKR = CK // 128
TBL = 16
QW = 32
NQ = 4
G_SEG = 64


def _round_up(v, m):
    return (v + m - 1) // m * m


def _edge_sweep(src_hbm, dst_hbm, tbl, acc, sidx, didx, r0, r1, sem0, sem1,
                row_base, groups):
    def fire(j, rbuf, sem):
        pltpu.async_copy(tbl.at[sidx.at[pl.ds(j * 256, 256)]], rbuf, sem)

    def wait(j, rbuf, sem):
        pltpu.make_async_copy(tbl.at[sidx.at[pl.ds(j * 256, 256)]], rbuf,
                              sem).wait()

    def scat(j, rbuf):
        pltpu.sync_copy(rbuf, acc.at[didx.at[pl.ds(j * 256, 256)]], add=True)

    @pl.loop(0, groups)
    def _(g):
        e0 = (row_base + g * 8) * 128
        pltpu.sync_copy(src_hbm.at[pl.ds(e0, 1024)], sidx)
        pltpu.sync_copy(dst_hbm.at[pl.ds(e0, 1024)], didx)
        fire(0, r0, sem0)

        @pl.loop(0, 2)
        def _(p):
            j0 = 2 * p
            fire(j0 + 1, r1, sem1)
            wait(j0, r0, sem0)
            scat(j0, r0)

            @pl.when(p < 1)
            def _():
                fire(j0 + 2, r0, sem0)

            wait(j0 + 1, r1, sem1)
            scat(j0 + 1, r1)


def _deg_kernel(npad, erows, interpret=False):
    rz = npad // NSUB
    chunks = erows // (NCORES * NSUB * CKR)
    mesh = plsc.VectorSubcoreMesh(core_axis_name="c", subcore_axis_name="s",
                                  num_cores=NCORES, num_subcores=NSUB)

    @functools.partial(
        pl.kernel,
        out_type=jax.ShapeDtypeStruct((NCORES * npad, TBL), F32),
        mesh=mesh,
        scratch_types=[
            pltpu.VMEM((CKR, 128), jnp.int32),
            pltpu.VMEM((128, TBL), F32),
            pltpu.VMEM_SHARED((npad, TBL), F32),
        ],
        compiler_params=pltpu.CompilerParams(use_tc_tiling_on_sc=False),
        interpret=interpret,
    )
    def k(dst_hbm, zeros_hbm, out_hbm, didx, ones_v, acc):
        c = lax.axis_index("c")
        s = lax.axis_index("s")
        wid = c * NSUB + s

        @pl.loop(0, 128)
        def _(r):
            ones_v[r, :] = jnp.full((TBL,), 1.0, F32)

        pltpu.sync_copy(zeros_hbm.at[pl.ds(s * rz, rz)],
                        acc.at[pl.ds(s * rz, rz)])
        plsc.subcore_barrier()

        @pl.loop(0, chunks)
        def _(kk):
            row0 = (wid * chunks + kk) * CKR
            pltpu.sync_copy(dst_hbm.at[pl.ds(row0, CKR)], didx)
            for j in range(CKR):
                pltpu.sync_copy(ones_v.at[pl.ds(0, 128)],
                                acc.at[didx.at[j]], add=True)

        plsc.subcore_barrier()
        pltpu.sync_copy(acc.at[pl.ds(s * rz, rz)],
                        out_hbm.at[pl.ds(c * npad + s * rz, rz)])

    return k


def _agg1_kernel(npad, erows, interpret=False):
    rz = npad // NSUB
    groups = erows // (NCORES * NSUB * 8)
    mesh = plsc.VectorSubcoreMesh(core_axis_name="c", subcore_axis_name="s",
                                  num_cores=NCORES, num_subcores=NSUB)

    @functools.partial(
        pl.kernel,
        out_type=jax.ShapeDtypeStruct((NCORES * npad, TBL), F32),
        mesh=mesh,
        scratch_types=[
            pltpu.VMEM((1024,), jnp.int32),
            pltpu.VMEM((1024,), jnp.int32),
            pltpu.VMEM((256, TBL), F32),
            pltpu.VMEM((256, TBL), F32),
            pltpu.VMEM_SHARED((npad, TBL), F32),
            pltpu.SemaphoreType.DMA,
            pltpu.SemaphoreType.DMA,
        ],
        compiler_params=pltpu.CompilerParams(use_tc_tiling_on_sc=False),
        interpret=interpret,
    )
    def k(src_hbm, dst_hbm, tbl_hbm, zeros_hbm, out_hbm, sidx, didx, r0, r1,
          acc, sem0, sem1):
        c = lax.axis_index("c")
        s = lax.axis_index("s")
        wid = c * NSUB + s

        pltpu.sync_copy(zeros_hbm.at[pl.ds(s * rz, rz)],
                        acc.at[pl.ds(s * rz, rz)])
        plsc.subcore_barrier()
        _edge_sweep(src_hbm, dst_hbm, tbl_hbm, acc, sidx, didx, r0, r1,
                    sem0, sem1, wid * groups * 8, groups)
        plsc.subcore_barrier()
        pltpu.sync_copy(acc.at[pl.ds(s * rz, rz)],
                        out_hbm.at[pl.ds(c * npad + s * rz, rz)])

    return k


def _agg2_kernel(npad, erows, interpret=False):
    rz = npad // NSUB
    groups = erows // (NSUB * 8)
    mesh = plsc.VectorSubcoreMesh(core_axis_name="c", subcore_axis_name="s",
                                  num_cores=NCORES, num_subcores=NSUB)

    @functools.partial(
        pl.kernel,
        out_type=jax.ShapeDtypeStruct((NQ * npad, QW), F32),
        mesh=mesh,
        scratch_types=[
            pltpu.VMEM((1024,), jnp.int32),
            pltpu.VMEM((1024,), jnp.int32),
            pltpu.VMEM((256, QW), F32),
            pltpu.VMEM((256, QW), F32),
            pltpu.VMEM_SHARED((npad, QW), F32),
            pltpu.SemaphoreType.DMA,
            pltpu.SemaphoreType.DMA,
        ],
        compiler_params=pltpu.CompilerParams(use_tc_tiling_on_sc=False),
        interpret=interpret,
    )
    def k(src_hbm, dst_hbm, g0, g1, g2, g3, zeros_hbm, out_hbm, sidx, didx,
          r0, r1, acc, sem0, sem1):
        c = lax.axis_index("c")
        s = lax.axis_index("s")
        gq_refs = [g0, g1, g2, g3]

        for q in range(NQ):
            gq = gq_refs[q]

            @pl.when(c == q // 2)
            def _():
                pltpu.sync_copy(zeros_hbm.at[pl.ds(s * rz, rz)],
                                acc.at[pl.ds(s * rz, rz)])
                plsc.subcore_barrier()
                _edge_sweep(src_hbm, dst_hbm, gq, acc, sidx, didx, r0,
                            r1, sem0, sem1, s * groups * 8, groups)
                plsc.subcore_barrier()
                pltpu.sync_copy(acc.at[pl.ds(s * rz, rz)],
                                out_hbm.at[pl.ds(q * npad + s * rz, rz)])
                plsc.subcore_barrier()

    return k


def _dinv_call(deg16, xp, npad, nfeat, interpret=False):
    bn = npad // NSUB

    def body(deg_ref, x_ref, dinv_ref, tbl_ref):
        deg = deg_ref[0, :, 0:1] + deg_ref[1, :, 0:1] + 1.0
        dinv = lax.rsqrt(deg)
        dinv_ref[...] = dinv
        gx = dinv * x_ref[...]
        tbl_ref[...] = jnp.concatenate(
            [gx, jnp.zeros((bn, TBL - nfeat), F32)], axis=1)

    return pl.pallas_call(
        body,
        grid=(NSUB,),
        in_specs=[
            pl.BlockSpec((NCORES, bn, TBL), lambda i: (0, i, 0)),
            pl.BlockSpec((bn, nfeat), lambda i: (i, 0)),
        ],
        out_specs=[
            pl.BlockSpec((bn, 1), lambda i: (i, 0)),
            pl.BlockSpec((bn, TBL), lambda i: (i, 0)),
        ],
        out_shape=[
            jax.ShapeDtypeStruct((npad, 1), F32),
            jax.ShapeDtypeStruct((npad, TBL), F32),
        ],
        interpret=interpret,
    )(deg16, xp)


def _dense_call(aggx, tbl, dinv, W1, b1, W2, npad, nfeat, interpret=False):
    bn = npad // NSUB

    def body(aggx_ref, tbl_ref, dinv_ref, w1_ref, b1_ref, w2_ref,
             g0_ref, g1_ref, g2_ref, g3_ref):
        dinv = dinv_ref[...]
        aggp = aggx_ref[0, :, 0:nfeat] + aggx_ref[1, :, 0:nfeat]
        ax = dinv * (aggp + tbl_ref[:, 0:nfeat])
        h1 = jnp.maximum(
            jnp.dot(ax, w1_ref[...], preferred_element_type=F32)
            + b1_ref[...], 0.0)
        m = jnp.dot(h1, w2_ref[...], preferred_element_type=F32)
        g = dinv * m
        g0_ref[...] = g[:, 0:QW]
        g1_ref[...] = g[:, QW:2 * QW]
        g2_ref[...] = g[:, 2 * QW:3 * QW]
        g3_ref[...] = g[:, 3 * QW:4 * QW]

    gspec = pl.BlockSpec((bn, QW), lambda i: (i, 0))
    gshape = jax.ShapeDtypeStruct((npad, QW), F32)
    return pl.pallas_call(
        body,
        grid=(NSUB,),
        in_specs=[
            pl.BlockSpec((NCORES, bn, TBL), lambda i: (0, i, 0)),
            pl.BlockSpec((bn, TBL), lambda i: (i, 0)),
            pl.BlockSpec((bn, 1), lambda i: (i, 0)),
            pl.BlockSpec((nfeat, 128), lambda i: (0, 0)),
            pl.BlockSpec((1, 128), lambda i: (0, 0)),
            pl.BlockSpec((128, 128), lambda i: (0, 0)),
        ],
        out_specs=[gspec, gspec, gspec, gspec],
        out_shape=[gshape, gshape, gshape, gshape],
        interpret=interpret,
    )(aggx, tbl, dinv, W1, b1, W2)


def _final_call(agg, g0, g1, g2, g3, dinv, b2, batch_row, Wfc, bfc, npad,
                interpret=False):
    bn = npad // NSUB
    nb = NSUB

    def body(a0_ref, a1_ref, a2_ref, a3_ref, g0_ref, g1_ref, g2_ref, g3_ref,
             dinv_ref, b2_ref, batch_ref, wfc_ref, bfc_ref, out_ref,
             pooled_acc, cnt_acc):
        i = pl.program_id(0)

        @pl.when(i == 0)
        def _():
            pooled_acc[...] = jnp.zeros((G_SEG, 128), F32)
            cnt_acc[...] = jnp.zeros((G_SEG, 128), F32)
            out_ref[...] = jnp.zeros((G_SEG, 10), F32)

        dinv = dinv_ref[...]
        h2 = jnp.concatenate(
            [jnp.maximum(dinv * (a_ref[...] + g_ref[...])
                         + b2_ref[:, q * QW:(q + 1) * QW], 0.0)
             for q, (a_ref, g_ref) in enumerate(
                 zip([a0_ref, a1_ref, a2_ref, a3_ref],
                     [g0_ref, g1_ref, g2_ref, g3_ref]))], axis=1)

        seg = batch_ref[...]
        onehot_t = (lax.broadcasted_iota(jnp.int32, (G_SEG, bn), 0)
                    == seg).astype(F32)
        pooled_acc[...] += lax.dot_general(
            onehot_t, h2, (((1,), (0,)), ((), ())),
            preferred_element_type=F32)
        cnt_acc[...] += lax.dot_general(
            onehot_t, jnp.ones((bn, 128), F32), (((1,), (0,)), ((), ())),
            preferred_element_type=F32)

        @pl.when(i == nb - 1)
        def _():
            mean = pooled_acc[...] / jnp.maximum(cnt_acc[...], 1.0)
            out_ref[...] = jnp.dot(mean, wfc_ref[...],
                                   preferred_element_type=F32) + bfc_ref[...]

    aspec = lambda q: pl.BlockSpec((bn, QW),
                                   lambda i, q=q: (q * NSUB + i, 0))
    gspec = pl.BlockSpec((bn, QW), lambda i: (i, 0))
    return pl.pallas_call(
        body,
        grid=(nb,),
        in_specs=[
            aspec(0), aspec(1), aspec(2), aspec(3),
            gspec, gspec, gspec, gspec,
            pl.BlockSpec((bn, 1), lambda i: (i, 0)),
            pl.BlockSpec((1, 128), lambda i: (0, 0)),
            pl.BlockSpec((1, bn), lambda i: (0, i)),
            pl.BlockSpec((128, 10), lambda i: (0, 0)),
            pl.BlockSpec((1, 10), lambda i: (0, 0)),
        ],
        out_specs=pl.BlockSpec((G_SEG, 10), lambda i: (0, 0)),
        out_shape=jax.ShapeDtypeStruct((G_SEG, 10), F32),
        scratch_shapes=[
            pltpu.VMEM((G_SEG, 128), F32),
            pltpu.VMEM((G_SEG, 128), F32),
        ],
        interpret=interpret,
    )(agg, agg, agg, agg, g0, g1, g2, g3, dinv, b2, batch_row, Wfc, bfc)


def _run(x, edge_index, batch, W1, b1, W2, b2, Wfc, bfc,
         interpret_sc=False, interpret_tc=False):
    n, nfeat = x.shape
    e = edge_index.shape[1]
    npad = _round_up(n + 1, NSUB * 128)
    epad = _round_up(e, NCORES * NSUB * 8 * 128)
    erows = epad // 128

    src = edge_index[0]
    dst = edge_index[1]
    padlen = epad - e
    fill = jnp.full((padlen,), n, jnp.int32)
    srcv = jnp.concatenate([src, fill])
    dstv = jnp.concatenate([dst, fill])
    dstr = dstv.reshape(erows, 128)
    xp = jnp.pad(x, ((0, npad - n), (0, 0)))
    batch_row = jnp.pad(batch, (0, npad - n),
                        constant_values=G_SEG).reshape(1, npad)
    zeros16 = jnp.zeros((npad, TBL), F32)
    zeros32 = jnp.zeros((npad, QW), F32)

    degp = _deg_kernel(npad, erows, interpret_sc)(dstr, zeros16)
    degp = degp.reshape(NCORES, npad, TBL)
    dinv, tbl = _dinv_call(degp, xp, npad, nfeat, interpret_tc)
    aggx = _agg1_kernel(npad, erows, interpret_sc)(srcv, dstv, tbl, zeros16)
    aggx = aggx.reshape(NCORES, npad, TBL)
    g0, g1, g2, g3 = _dense_call(aggx, tbl, dinv, W1, b1.reshape(1, 128), W2,
                                 npad, nfeat, interpret_tc)
    agg = _agg2_kernel(npad, erows, interpret_sc)(srcv, dstv, g0, g1, g2, g3,
                                                  zeros32)
    out = _final_call(agg, g0, g1, g2, g3, dinv, b2.reshape(1, 128),
                      batch_row, Wfc, bfc.reshape(1, 10), npad, interpret_tc)
    return out


def kernel(x, edge_index, batch, W1, b1, W2, b2, Wfc, bfc):
    return _run(x, edge_index, batch, W1, b1, W2, b2, Wfc, bfc)
